# baseline (speedup 1.0000x reference)
import jax
import jax.numpy as jnp
from jax import lax
from jax.experimental import pallas as pl
from jax.experimental.pallas import tpu as pltpu
from jax.experimental.pallas import tpu_sc as plsc

B = 4096
S = 200
D = 64
DP = 2 * D
HIDDEN = 300
VOCAB_ROWS = 1000000
HALF = VOCAB_ROWS // 2

NC = 2
NS = 16
NW = NC * NS
SAMP_PER_W = B // NW
CHUNK = S // 2
CHUNKS_PER_W = SAMP_PER_W * 2
NBUF = 4

_sc_mesh = plsc.VectorSubcoreMesh(
    core_axis_name="c", subcore_axis_name="s", num_cores=NC, num_subcores=NS
)


_CBLK = 4000


def _compact_body(a_ref, b_ref, o_ref):
    o_ref[:, 0:D] = a_ref[...]
    o_ref[:, D:DP] = b_ref[...]


def _compact(table):
    nblk = HALF // _CBLK
    return pl.pallas_call(
        _compact_body,
        grid=(nblk,),
        in_specs=[
            pl.BlockSpec((_CBLK, D), lambda i: (i, 0)),
            pl.BlockSpec((_CBLK, D), lambda i, nblk=nblk: (i + nblk, 0)),
        ],
        out_specs=pl.BlockSpec((_CBLK, DP), lambda i: (i, 0)),
        out_shape=jax.ShapeDtypeStruct((HALF, DP), jnp.float32),
    )(table, table)


def _pool_body(x_hbm, off_hbm, table_hbm, out_hbm, idx_v, rows_v,
               out_v, off_v, *sems):
    w = lax.axis_index("s") * NC + lax.axis_index("c")
    rsems, osems = sems[:NBUF], sems[NBUF:]

    pltpu.sync_copy(x_hbm.at[w], idx_v)

    def gather(g, b):
        pltpu.async_copy(
            table_hbm.at[idx_v.at[g, pl.ds(0, CHUNK)]], rows_v.at[b], rsems[b]
        )
        pltpu.async_copy(off_hbm.at[w, g], off_v.at[b], osems[b])

    for b in range(NBUF):
        gather(b, b)

    def outer(it, carry):
        for half in range(NBUF // 2):
            smp = it * (NBUF // 2) + half
            accs = tuple(jnp.zeros((16,), jnp.float32) for _ in range(4))
            for p in range(2):
                b = half * 2 + p
                g = it * NBUF + b
                pltpu.make_async_copy(
                    table_hbm.at[idx_v.at[0, pl.ds(0, CHUNK)]],
                    rows_v.at[b],
                    rsems[b],
                ).wait()
                pltpu.make_async_copy(
                    off_hbm.at[w, 0], off_v.at[b], osems[b]
                ).wait()

                def grp_body(g2, a, b=b):
                    off16 = off_v[b, pl.ds(g2 * 16, 16)]
                    for j in range(16):
                        oj = off16[j]
                        r = g2 * 16 + j
                        a = tuple(
                            a[k] + rows_v[b, r, pl.ds(oj + 16 * k, 16)]
                            for k in range(4)
                        )
                    return a

                accs = lax.fori_loop(0, CHUNK // 16, grp_body, accs)
                off16t = off_v[b, pl.ds(16 * (CHUNK // 16), 16)]
                for j in range(CHUNK % 16):
                    oj = off16t[j]
                    r = 16 * (CHUNK // 16) + j
                    accs = tuple(
                        accs[k] + rows_v[b, r, pl.ds(oj + 16 * k, 16)]
                        for k in range(4)
                    )

                g_next = (it + 1) * NBUF + b

                @pl.when(g_next < CHUNKS_PER_W)
                def _(b=b, g_next=g_next):
                    gather(g_next, b)

            inv = jnp.float32(1.0 / S)
            for k in range(4):
                out_v[smp, pl.ds(16 * k, 16)] = accs[k] * inv
        return carry

    lax.fori_loop(0, CHUNKS_PER_W // NBUF, outer, 0)

    pltpu.sync_copy(out_v, out_hbm.at[pl.ds(w * SAMP_PER_W, SAMP_PER_W)])


_sc_pool = pl.kernel(
    _pool_body,
    out_type=jax.ShapeDtypeStruct((B, D), jnp.float32),
    mesh=_sc_mesh,
    scratch_types=[
        pltpu.VMEM((CHUNKS_PER_W, DP), jnp.int32),
        pltpu.VMEM((NBUF, CHUNK, DP), jnp.float32),
        pltpu.VMEM((SAMP_PER_W, D), jnp.float32),
        pltpu.VMEM((NBUF, DP), jnp.int32),
    ]
    + [pltpu.SemaphoreType.DMA] * (2 * NBUF),
)


def _mlp_body(x_ref, w1_ref, b1_ref, w2_ref, b2_ref, w3_ref, b3_ref, o_ref):
    x = x_ref[...]
    h = jnp.maximum(
        lax.dot_general(
            x, w1_ref[...], (((1,), (0,)), ((), ())),
            preferred_element_type=jnp.float32,
        )
        + b1_ref[...],
        0.0,
    )
    h = jnp.maximum(
        lax.dot_general(
            h, w2_ref[...], (((1,), (0,)), ((), ())),
            preferred_element_type=jnp.float32,
        )
        + b2_ref[...],
        0.0,
    )
    logits = (
        lax.dot_general(
            h, w3_ref[...], (((1,), (0,)), ((), ())),
            preferred_element_type=jnp.float32,
        )
        + b3_ref[...]
    )
    m = jnp.max(logits, axis=1, keepdims=True)
    lse = m + jnp.log(jnp.sum(jnp.exp(logits - m), axis=1, keepdims=True))
    o_ref[...] = logits - lse


_MLP_BB = 512


def _mlp(pooled, W1, b1, W2, b2, W3, b3):
    grid = (B // _MLP_BB,)
    return pl.pallas_call(
        _mlp_body,
        grid=grid,
        in_specs=[
            pl.BlockSpec((_MLP_BB, D), lambda i: (i, 0)),
            pl.BlockSpec((D, HIDDEN), lambda i: (0, 0)),
            pl.BlockSpec((1, HIDDEN), lambda i: (0, 0)),
            pl.BlockSpec((HIDDEN, HIDDEN), lambda i: (0, 0)),
            pl.BlockSpec((1, HIDDEN), lambda i: (0, 0)),
            pl.BlockSpec((HIDDEN, 2), lambda i: (0, 0)),
            pl.BlockSpec((1, 2), lambda i: (0, 0)),
        ],
        out_specs=pl.BlockSpec((_MLP_BB, 2), lambda i: (i, 0)),
        out_shape=jax.ShapeDtypeStruct((B, 2), jnp.float32),
    )(pooled, W1, b1, W2, b2, W3, b3)


def kernel(x, table, W1, b1, W2, b2, W3, b3):
    table_c = _compact(table)
    sel = (x >= HALF).astype(jnp.int32)
    xg = x - HALF * sel
    xoff = sel * D
    pad3 = ((0, 0), (0, 0), (0, DP - CHUNK))
    xr = jnp.pad(xg.reshape(NW, CHUNKS_PER_W, CHUNK), pad3)
    xo = jnp.pad(xoff.reshape(NW, CHUNKS_PER_W, CHUNK), pad3)
    pooled = _sc_pool(xr, xo, table_c)
    return _mlp(
        pooled, W1, b1.reshape(1, HIDDEN), W2, b2.reshape(1, HIDDEN),
        W3, b3.reshape(1, 2),
    )


# speedup vs baseline: 1.4900x; 1.4900x over previous
import jax
import jax.numpy as jnp
from jax import lax
from jax.experimental import pallas as pl
from jax.experimental.pallas import tpu as pltpu
from jax.experimental.pallas import tpu_sc as plsc

B = 4096
S = 200
D = 64
DP = 2 * D
HIDDEN = 300
VOCAB_ROWS = 1000000

NC = 2
NS = 16
NW = NC * NS
SAMP_PER_W = B // NW
CHUNK = S // 2
CHUNKS_PER_W = SAMP_PER_W * 2
NBUF = 4

_sc_mesh = plsc.VectorSubcoreMesh(
    core_axis_name="c", subcore_axis_name="s", num_cores=NC, num_subcores=NS
)


_CBLK = 4000


def _padk_body(a_ref, o_ref):
    o_ref[:, 0:D] = a_ref[...]
    o_ref[:, D:DP] = jnp.zeros((_CBLK, D), jnp.float32)


def _padk(table):
    return pl.pallas_call(
        _padk_body,
        grid=(VOCAB_ROWS // _CBLK,),
        in_specs=[pl.BlockSpec((_CBLK, D), lambda i: (i, 0))],
        out_specs=pl.BlockSpec((_CBLK, DP), lambda i: (i, 0)),
        out_shape=jax.ShapeDtypeStruct((VOCAB_ROWS, DP), jnp.float32),
    )(table)


def _pool_body(x_hbm, table_hbm, out_hbm, idx_v, rows_v, out_v, *sems):
    w = lax.axis_index("s") * NC + lax.axis_index("c")

    pltpu.sync_copy(x_hbm.at[w], idx_v)

    def gather(g, b):
        pltpu.async_copy(
            table_hbm.at[idx_v.at[g, pl.ds(0, CHUNK)]], rows_v.at[b], sems[b]
        )

    for b in range(NBUF):
        gather(b, b)

    def outer(it, carry):
        for half in range(NBUF // 2):
            smp = it * (NBUF // 2) + half
            accs = tuple(jnp.zeros((16,), jnp.float32) for _ in range(4))
            for p in range(2):
                b = half * 2 + p
                pltpu.make_async_copy(
                    table_hbm.at[idx_v.at[0, pl.ds(0, CHUNK)]],
                    rows_v.at[b],
                    sems[b],
                ).wait()

                def row_body(r, a, b=b):
                    return tuple(
                        a[k] + rows_v[b, r, pl.ds(16 * k, 16)] for k in range(4)
                    )

                accs = lax.fori_loop(0, CHUNK, row_body, accs, unroll=4)

                g_next = (it + 1) * NBUF + b

                @pl.when(g_next < CHUNKS_PER_W)
                def _(b=b, g_next=g_next):
                    gather(g_next, b)

            inv = jnp.float32(1.0 / S)
            for k in range(4):
                out_v[smp, pl.ds(16 * k, 16)] = accs[k] * inv
        return carry

    lax.fori_loop(0, CHUNKS_PER_W // NBUF, outer, 0)

    pltpu.sync_copy(out_v, out_hbm.at[pl.ds(w * SAMP_PER_W, SAMP_PER_W)])


_sc_pool = pl.kernel(
    _pool_body,
    out_type=jax.ShapeDtypeStruct((B, D), jnp.float32),
    mesh=_sc_mesh,
    scratch_types=[
        pltpu.VMEM((CHUNKS_PER_W, DP), jnp.int32),
        pltpu.VMEM((NBUF, CHUNK, DP), jnp.float32),
        pltpu.VMEM((SAMP_PER_W, D), jnp.float32),
    ]
    + [pltpu.SemaphoreType.DMA] * NBUF,
)


def _mlp_body(x_ref, w1_ref, b1_ref, w2_ref, b2_ref, w3_ref, b3_ref, o_ref):
    x = x_ref[...]
    h = jnp.maximum(
        lax.dot_general(
            x, w1_ref[...], (((1,), (0,)), ((), ())),
            preferred_element_type=jnp.float32,
        )
        + b1_ref[...],
        0.0,
    )
    h = jnp.maximum(
        lax.dot_general(
            h, w2_ref[...], (((1,), (0,)), ((), ())),
            preferred_element_type=jnp.float32,
        )
        + b2_ref[...],
        0.0,
    )
    logits = (
        lax.dot_general(
            h, w3_ref[...], (((1,), (0,)), ((), ())),
            preferred_element_type=jnp.float32,
        )
        + b3_ref[...]
    )
    m = jnp.max(logits, axis=1, keepdims=True)
    lse = m + jnp.log(jnp.sum(jnp.exp(logits - m), axis=1, keepdims=True))
    o_ref[...] = logits - lse


_MLP_BB = 512


def _mlp(pooled, W1, b1, W2, b2, W3, b3):
    grid = (B // _MLP_BB,)
    return pl.pallas_call(
        _mlp_body,
        grid=grid,
        in_specs=[
            pl.BlockSpec((_MLP_BB, D), lambda i: (i, 0)),
            pl.BlockSpec((D, HIDDEN), lambda i: (0, 0)),
            pl.BlockSpec((1, HIDDEN), lambda i: (0, 0)),
            pl.BlockSpec((HIDDEN, HIDDEN), lambda i: (0, 0)),
            pl.BlockSpec((1, HIDDEN), lambda i: (0, 0)),
            pl.BlockSpec((HIDDEN, 2), lambda i: (0, 0)),
            pl.BlockSpec((1, 2), lambda i: (0, 0)),
        ],
        out_specs=pl.BlockSpec((_MLP_BB, 2), lambda i: (i, 0)),
        out_shape=jax.ShapeDtypeStruct((B, 2), jnp.float32),
    )(pooled, W1, b1, W2, b2, W3, b3)


def kernel(x, table, W1, b1, W2, b2, W3, b3):
    table_p = _padk(table)
    xr = jnp.pad(
        x.reshape(NW, CHUNKS_PER_W, CHUNK), ((0, 0), (0, 0), (0, DP - CHUNK))
    )
    pooled = _sc_pool(xr, table_p)
    return _mlp(
        pooled, W1, b1.reshape(1, HIDDEN), W2, b2.reshape(1, HIDDEN),
        W3, b3.reshape(1, 2),
    )


# speedup vs baseline: 1.5176x; 1.0185x over previous
import jax
import jax.numpy as jnp
from jax import lax
from jax.experimental import pallas as pl
from jax.experimental.pallas import tpu as pltpu
from jax.experimental.pallas import tpu_sc as plsc

B = 4096
S = 200
D = 64
DP = 2 * D
HIDDEN = 300
VOCAB_ROWS = 1000000

NC = 2
NS = 16
NW = NC * NS
SAMP_PER_W = B // NW
CHUNK = S // 2
CHUNKS_PER_W = SAMP_PER_W * 2
NBUF = 4

_sc_mesh = plsc.VectorSubcoreMesh(
    core_axis_name="c", subcore_axis_name="s", num_cores=NC, num_subcores=NS
)


_CBLK = 4000


def _padk_body(a_ref, o_ref):
    o_ref[:, 0:D] = a_ref[...]
    o_ref[:, D:DP] = jnp.zeros((_CBLK, D), jnp.float32)


def _padk(table):
    return pl.pallas_call(
        _padk_body,
        grid=(VOCAB_ROWS // _CBLK,),
        in_specs=[pl.BlockSpec((_CBLK, D), lambda i: (i, 0))],
        out_specs=pl.BlockSpec((_CBLK, DP), lambda i: (i, 0)),
        out_shape=jax.ShapeDtypeStruct((VOCAB_ROWS, DP), jnp.float32),
    )(table)


def _pool_body(x_hbm, table_hbm, out_hbm, idx_v, rows_v, out_v, *sems):
    w = lax.axis_index("s") * NC + lax.axis_index("c")

    pltpu.sync_copy(x_hbm.at[w], idx_v)

    def gather(g, b):
        pltpu.async_copy(
            table_hbm.at[idx_v.at[g, pl.ds(0, CHUNK)]], rows_v.at[b], sems[b]
        )

    for b in range(NBUF):
        gather(b, b)

    def outer(it, carry):
        for half in range(NBUF // 2):
            smp = it * (NBUF // 2) + half
            accs = tuple(jnp.zeros((16,), jnp.float32) for _ in range(4))
            for p in range(2):
                b = half * 2 + p
                pltpu.make_async_copy(
                    table_hbm.at[idx_v.at[0, pl.ds(0, CHUNK)]],
                    rows_v.at[b],
                    sems[b],
                ).wait()

                def row_body(r, a, b=b):
                    return tuple(
                        a[k] + rows_v[b, r, pl.ds(16 * k, 16)] for k in range(4)
                    )

                accs = lax.fori_loop(0, CHUNK, row_body, accs, unroll=4)

                g_next = (it + 1) * NBUF + b

                @pl.when(g_next < CHUNKS_PER_W)
                def _(b=b, g_next=g_next):
                    gather(g_next, b)

            inv = jnp.float32(1.0 / S)
            for k in range(4):
                out_v[smp, pl.ds(16 * k, 16)] = accs[k] * inv
        return carry

    lax.fori_loop(0, CHUNKS_PER_W // NBUF, outer, 0)

    pltpu.sync_copy(out_v, out_hbm.at[pl.ds(w * SAMP_PER_W, SAMP_PER_W)])


_sc_pool = pl.kernel(
    _pool_body,
    out_type=jax.ShapeDtypeStruct((B, D), jnp.float32),
    mesh=_sc_mesh,
    scratch_types=[
        pltpu.VMEM((CHUNKS_PER_W, DP), jnp.int32),
        pltpu.VMEM((NBUF, CHUNK, DP), jnp.float32),
        pltpu.VMEM((SAMP_PER_W, D), jnp.float32),
    ]
    + [pltpu.SemaphoreType.DMA] * NBUF,
)


def _mlp_body(x_ref, w1_ref, b1_ref, w2_ref, b2_ref, w3_ref, b3_ref, o_ref):
    x = x_ref[...]
    h = jnp.maximum(
        lax.dot_general(
            x, w1_ref[...], (((1,), (0,)), ((), ())),
            preferred_element_type=jnp.float32,
        )
        + b1_ref[...],
        0.0,
    )
    h = jnp.maximum(
        lax.dot_general(
            h, w2_ref[...], (((1,), (0,)), ((), ())),
            preferred_element_type=jnp.float32,
        )
        + b2_ref[...],
        0.0,
    )
    logits = (
        lax.dot_general(
            h, w3_ref[...], (((1,), (0,)), ((), ())),
            preferred_element_type=jnp.float32,
        )
        + b3_ref[...]
    )
    m = jnp.max(logits, axis=1, keepdims=True)
    lse = m + jnp.log(jnp.sum(jnp.exp(logits - m), axis=1, keepdims=True))
    o_ref[...] = logits - lse


_MLP_BB = 512


def _mlp(pooled, W1, b1, W2, b2, W3, b3):
    grid = (B // _MLP_BB,)
    return pl.pallas_call(
        _mlp_body,
        grid=grid,
        in_specs=[
            pl.BlockSpec((_MLP_BB, D), lambda i: (i, 0)),
            pl.BlockSpec((D, HIDDEN), lambda i: (0, 0)),
            pl.BlockSpec((1, HIDDEN), lambda i: (0, 0)),
            pl.BlockSpec((HIDDEN, HIDDEN), lambda i: (0, 0)),
            pl.BlockSpec((1, HIDDEN), lambda i: (0, 0)),
            pl.BlockSpec((HIDDEN, 2), lambda i: (0, 0)),
            pl.BlockSpec((1, 2), lambda i: (0, 0)),
        ],
        out_specs=pl.BlockSpec((_MLP_BB, 2), lambda i: (i, 0)),
        out_shape=jax.ShapeDtypeStruct((B, 2), jnp.float32),
    )(pooled, W1, b1, W2, b2, W3, b3)


def kernel(x, table, W1, b1, W2, b2, W3, b3):
    table_p = jnp.concatenate((table, table), axis=1)
    xr = jnp.pad(
        x.reshape(NW, CHUNKS_PER_W, CHUNK), ((0, 0), (0, 0), (0, DP - CHUNK))
    )
    pooled = _sc_pool(xr, table_p)
    return _mlp(
        pooled, W1, b1.reshape(1, HIDDEN), W2, b2.reshape(1, HIDDEN),
        W3, b3.reshape(1, 2),
    )
